# Initial kernel scaffold; baseline (speedup 1.0000x reference)
import jax
import jax.numpy as jnp
from jax.experimental import pallas as pl

_CAPACITY_FACTOR = 1.25
_BT = 2048


def _matmul_body(x_ref, w_ref, out_ref):
    out_ref[...] = jax.lax.dot_general(
        x_ref[...], w_ref[...],
        dimension_numbers=(((1,), (1,)), ((), ())),
        preferred_element_type=jnp.float32,
    )


def kernel(x, W):
    T, H = x.shape
    E = W.shape[0]
    k = int(T * _CAPACITY_FACTOR / E)
    logits = pl.pallas_call(
        _matmul_body,
        grid=(T // _BT,),
        in_specs=[
            pl.BlockSpec((_BT, H), lambda i: (i, 0)),
            pl.BlockSpec((E, H), lambda i: (0, 0)),
        ],
        out_specs=pl.BlockSpec((_BT, E), lambda i: (i, 0)),
        out_shape=jax.ShapeDtypeStruct((T, E), jnp.float32),
    )(x, W)
    _, expert_indices = jax.lax.top_k(logits.T, k)
    rows = expert_indices.T.reshape(-1)
    cols = jnp.tile(jnp.arange(E), k)
    dispatch_mask = jnp.zeros_like(logits).at[rows, cols].set(1.0)
    expert_load = dispatch_mask.sum(axis=0)
    loss = (expert_load * jnp.log(expert_load / expert_load.mean())).mean()
    return expert_indices, dispatch_mask, loss


# speedup vs baseline: 1.0005x; 1.0005x over previous
import jax
import jax.numpy as jnp
from jax.experimental import pallas as pl

_CAPACITY_FACTOR = 1.25
_BT = 2048


def _matmul_body(x_ref, w_ref, out_ref):
    out_ref[...] = jax.lax.dot_general(
        w_ref[...], x_ref[...],
        dimension_numbers=(((1,), (1,)), ((), ())),
        preferred_element_type=jnp.float32,
    )


def kernel(x, W):
    T, H = x.shape
    E = W.shape[0]
    k = int(T * _CAPACITY_FACTOR / E)
    logitsT = pl.pallas_call(
        _matmul_body,
        grid=(T // _BT,),
        in_specs=[
            pl.BlockSpec((_BT, H), lambda i: (i, 0)),
            pl.BlockSpec((E, H), lambda i: (0, 0)),
        ],
        out_specs=pl.BlockSpec((E, _BT), lambda i: (0, i)),
        out_shape=jax.ShapeDtypeStruct((E, T), jnp.float32),
    )(x, W)
    _, expert_indices = jax.lax.top_k(logitsT, k)
    rows = expert_indices.T.reshape(-1)
    cols = jnp.tile(jnp.arange(E), k)
    dispatch_mask = jnp.zeros((T, E), jnp.float32).at[rows, cols].set(1.0)
    expert_load = dispatch_mask.sum(axis=0)
    loss = (expert_load * jnp.log(expert_load / expert_load.mean())).mean()
    return expert_indices, dispatch_mask, loss


# speedup vs baseline: 4.0819x; 4.0798x over previous
import functools

import jax
import jax.numpy as jnp
from jax.experimental import pallas as pl
from jax.experimental.pallas import tpu as pltpu
from jax.experimental.pallas import tpu_sc as plsc

_CAPACITY_FACTOR = 1.25
_BT = 2048
_BD = 4096
_NSURV = 1024
_SPAD = 1040
_PAD_KEY = -2147483648
_PAD_IDX = 0x7FFFFFF


def _matmul_body(x_ref, w_ref, out_ref):
    out_ref[...] = jax.lax.dot_general(
        w_ref[...], x_ref[...],
        dimension_numbers=(((1,), (1,)), ((), ())),
        preferred_element_type=jnp.float32,
    )


def _logits_t(x, W, T, H, E):
    return pl.pallas_call(
        _matmul_body,
        grid=(T // _BT,),
        in_specs=[
            pl.BlockSpec((_BT, H), lambda i: (i, 0)),
            pl.BlockSpec((E, H), lambda i: (0, 0)),
        ],
        out_specs=pl.BlockSpec((E, _BT), lambda i: (0, i)),
        out_shape=jax.ShapeDtypeStruct((E, T), jnp.float32),
    )(x, W)


def _mono_key_tc(f):
    u = jax.lax.bitcast_convert_type(f, jnp.int32)
    return jnp.where(u >= 0, u, u ^ jnp.int32(0x7FFFFFFF))


def _cmp_before(ka, ia, kb, ib):
    return (ka > kb) | ((ka == kb) & (ia < ib))


def _sort_body(k, skeys_ref, sidx_ref, ei_ref, cutk_ref, cuti_ref):
    keys = skeys_ref[:, :_NSURV]
    idxs = sidx_ref[:, :_NSURV]
    E = keys.shape[0]
    pos = jax.lax.broadcasted_iota(jnp.int32, (E, _NSURV), 1)
    size = 2
    while size <= _NSURV:
        want_desc = (pos & size) == 0 if size < _NSURV else (pos >= 0)
        j = size // 2
        while j >= 1:
            self_first = (pos & j) == 0
            pk = jnp.where(self_first, jnp.roll(keys, -j, axis=1),
                           jnp.roll(keys, j, axis=1))
            pi = jnp.where(self_first, jnp.roll(idxs, -j, axis=1),
                           jnp.roll(idxs, j, axis=1))
            winner_is_self = _cmp_before(keys, idxs, pk, pi)
            take_self = (want_desc == self_first) == winner_is_self
            keys = jnp.where(take_self, keys, pk)
            idxs = jnp.where(take_self, idxs, pi)
            j //= 2
        size *= 2
    ei_ref[...] = idxs[:, :k]
    cutk_ref[...] = jnp.broadcast_to(keys[:, k - 1:k], (E, 128))
    cuti_ref[...] = jnp.broadcast_to(idxs[:, k - 1:k], (E, 128))


def _sort_survivors(skeys, sidx, E, k):
    return pl.pallas_call(
        functools.partial(_sort_body, k),
        in_specs=[
            pl.BlockSpec((E, _SPAD), lambda: (0, 0)),
            pl.BlockSpec((E, _SPAD), lambda: (0, 0)),
        ],
        out_specs=[
            pl.BlockSpec((E, k), lambda: (0, 0)),
            pl.BlockSpec((E, 128), lambda: (0, 0)),
            pl.BlockSpec((E, 128), lambda: (0, 0)),
        ],
        out_shape=[
            jax.ShapeDtypeStruct((E, k), jnp.int32),
            jax.ShapeDtypeStruct((E, 128), jnp.int32),
            jax.ShapeDtypeStruct((E, 128), jnp.int32),
        ],
    )(skeys, sidx)


def _mask_body(k, logits_ref, cutk_ref, cuti_ref, mask_ref, loss_ref,
               acc_ref):
    i = pl.program_id(0)
    E, BT = logits_ref.shape
    key = _mono_key_tc(logits_ref[...])
    tok = jax.lax.broadcasted_iota(jnp.int32, (E, BT), 1) + i * BT
    cutk = cutk_ref[:, :1]
    cuti = cuti_ref[:, :1]
    sel = (key > cutk) | ((key == cutk) & (tok <= cuti))
    m = jnp.where(sel, jnp.float32(1.0), jnp.float32(0.0))
    mask_ref[...] = m.T

    @pl.when(i == 0)
    def _():
        acc_ref[...] = jnp.zeros_like(acc_ref)

    acc_ref[0:1, :E] += jnp.sum(m, axis=1)[None, :]

    @pl.when(i == pl.num_programs(0) - 1)
    def _():
        load = acc_ref[0:1, :E]
        loss = jnp.mean(load * jnp.log(load / jnp.mean(load)))
        loss_ref[...] = jnp.full_like(loss_ref, loss)


def _dispatch_mask(logitsT, cutk, cuti, T, E, k):
    mask, loss = pl.pallas_call(
        functools.partial(_mask_body, k),
        grid=(T // _BD,),
        in_specs=[
            pl.BlockSpec((E, _BD), lambda i: (0, i)),
            pl.BlockSpec((E, 128), lambda i: (0, 0)),
            pl.BlockSpec((E, 128), lambda i: (0, 0)),
        ],
        out_specs=[
            pl.BlockSpec((_BD, E), lambda i: (i, 0)),
            pl.BlockSpec((8, 128), lambda i: (0, 0)),
        ],
        out_shape=[
            jax.ShapeDtypeStruct((T, E), jnp.float32),
            jax.ShapeDtypeStruct((8, 128), jnp.float32),
        ],
        scratch_shapes=[pltpu.VMEM((8, 128), jnp.float32)],
    )(logitsT, cutk, cuti)
    return mask, loss[0, 0]


def _mono_key_sc(f):
    u = jax.lax.bitcast_convert_type(f, jnp.int32)
    return jnp.where(u >= 0, u, u ^ jnp.int32(0x7FFFFFFF))


def _select_body(T, E, k, logits_hbm, skeys_hbm, sidx_hbm,
                 row_v, keys_v, hist_v, skeys_v, sidx_v):
    nvec = T // 16
    nhist = 2048
    lane = jax.lax.iota(jnp.int32, 16)
    ones = jnp.ones((16,), jnp.int32)
    zeros = jnp.zeros((16,), jnp.int32)
    wid = jax.lax.axis_index("s") * 2 + jax.lax.axis_index("c")

    def hist_pass(bin_mask_fn):
        def z(i, c):
            hist_v[pl.ds(i * 16, 16)] = zeros
            return c
        jax.lax.fori_loop(0, nhist, z, 0)

        def h(i, c):
            key = keys_v[pl.ds(i * 16, 16)]
            b, m = bin_mask_fn(key)
            plsc.addupdate_scatter(hist_v, [lane * nhist + b], ones, mask=m)
            return c
        jax.lax.fori_loop(0, nvec, h, 0)

    def find_bucket(rem):
        def body(jj, carry):
            above, bstar, cnt_at = carry
            j = 127 - jj
            tot = zeros
            for l in range(16):
                tot = tot + hist_v[pl.ds(l * nhist + j * 16, 16)]
            suf = jax.lax.rev(plsc.cumsum(jax.lax.rev(tot, (0,))), (0,))
            cum_gt = above + suf - tot
            hit = (cum_gt < rem) & (cum_gt + tot >= rem)
            hitn = jax.lax.reduce_max(jnp.where(hit, ones, zeros), axes=(0,))
            bsel = jax.lax.reduce_max(
                jnp.where(hit, j * 16 + lane, jnp.full((16,), -1, jnp.int32)),
                axes=(0,))
            csel = jax.lax.reduce_max(
                jnp.where(hit, cum_gt, jnp.full((16,), -1, jnp.int32)),
                axes=(0,))
            bstar = jnp.where(hitn > 0, bsel, bstar)
            cnt_at = jnp.where(hitn > 0, csel, cnt_at)
            above = above + jax.lax.reduce_sum(tot, axes=(0,))
            return above, bstar, cnt_at
        _, bstar, cnt_at = jax.lax.fori_loop(
            0, 128, body, (jnp.int32(0), jnp.int32(-1), jnp.int32(0)))
        return bstar, cnt_at

    for e in range(2):
        r = wid * 2 + e
        pltpu.sync_copy(logits_hbm.at[r], row_v)

        def mk(i, c):
            keys_v[pl.ds(i * 16, 16)] = _mono_key_sc(row_v[pl.ds(i * 16, 16)])
            return c
        jax.lax.fori_loop(0, nvec, mk, 0)

        hist_pass(lambda key: ((key >> 21) + 1024, None))
        b1, c1 = find_bucket(jnp.int32(k))
        hist_pass(lambda key: ((key >> 10) & jnp.int32(0x7FF),
                               ((key >> 21) + 1024) == b1))
        b2, c2 = find_bucket(jnp.int32(k) - c1)
        pref = ((b1 - 1024) << 11) | b2
        hist_pass(lambda key: (key & jnp.int32(0x3FF), (key >> 10) == pref))
        b3, _ = find_bucket(jnp.int32(k) - c1 - c2)
        t = ((b1 - 1024) << 21) | (b2 << 10) | b3

        def zpad(i, c):
            skeys_v[pl.ds(i * 16, 16)] = jnp.full((16,), _PAD_KEY, jnp.int32)
            sidx_v[pl.ds(i * 16, 16)] = jnp.full((16,), _PAD_IDX, jnp.int32)
            return c
        jax.lax.fori_loop(0, _SPAD // 16, zpad, 0)

        def comp(i, off):
            key = keys_v[pl.ds(i * 16, 16)]
            m = key >= t
            cnt = jax.lax.reduce_sum(jnp.where(m, ones, zeros), axes=(0,))

            @pl.when(off < _NSURV)
            def _():
                plsc.store_compressed(skeys_v.at[pl.ds(off, 16)], key, mask=m)
                plsc.store_compressed(sidx_v.at[pl.ds(off, 16)],
                                      i * 16 + lane, mask=m)
            return off + cnt
        jax.lax.fori_loop(0, nvec, comp, jnp.int32(0))

        pltpu.sync_copy(skeys_v, skeys_hbm.at[r])
        pltpu.sync_copy(sidx_v, sidx_hbm.at[r])


def _select_survivors_sc(logitsT, E, T, k):
    mesh = plsc.VectorSubcoreMesh(core_axis_name="c", subcore_axis_name="s")
    f = pl.kernel(
        functools.partial(_select_body, T, E, k),
        out_type=[
            jax.ShapeDtypeStruct((E, _SPAD), jnp.int32),
            jax.ShapeDtypeStruct((E, _SPAD), jnp.int32),
        ],
        mesh=mesh,
        scratch_types=[
            pltpu.VMEM((T,), jnp.float32),
            pltpu.VMEM((T,), jnp.int32),
            pltpu.VMEM((T,), jnp.int32),
            pltpu.VMEM((_SPAD,), jnp.int32),
            pltpu.VMEM((_SPAD,), jnp.int32),
        ],
        compiler_params=pltpu.CompilerParams(needs_layout_passes=False),
    )
    return f(logitsT)


def kernel(x, W):
    T, H = x.shape
    E = W.shape[0]
    k = int(T * _CAPACITY_FACTOR / E)
    logitsT = _logits_t(x, W, T, H, E)
    skeys, sidx = _select_survivors_sc(logitsT, E, T, k)
    expert_indices, cutk, cuti = _sort_survivors(skeys, sidx, E, k)
    dispatch_mask, loss = _dispatch_mask(logitsT, cutk, cuti, T, E, k)
    return expert_indices, dispatch_mask, loss


# speedup vs baseline: 5.0475x; 1.2365x over previous
import functools

import jax
import jax.numpy as jnp
from jax.experimental import pallas as pl
from jax.experimental.pallas import tpu as pltpu
from jax.experimental.pallas import tpu_sc as plsc

_CAPACITY_FACTOR = 1.25
_BT = 2048
_BD = 4096
_NSURV = 1024
_SPAD = 1040
_PAD_KEY = -2147483648
_PAD_IDX = 0x7FFFFFF


def _matmul_body(x_ref, w_ref, out_ref):
    out_ref[...] = jax.lax.dot_general(
        w_ref[...], x_ref[...],
        dimension_numbers=(((1,), (1,)), ((), ())),
        preferred_element_type=jnp.float32,
    )


def _logits_t(x, W, T, H, E):
    return pl.pallas_call(
        _matmul_body,
        grid=(T // _BT,),
        in_specs=[
            pl.BlockSpec((_BT, H), lambda i: (i, 0)),
            pl.BlockSpec((E, H), lambda i: (0, 0)),
        ],
        out_specs=pl.BlockSpec((E, _BT), lambda i: (0, i)),
        out_shape=jax.ShapeDtypeStruct((E, T), jnp.float32),
    )(x, W)


def _mono_key_tc(f):
    u = jax.lax.bitcast_convert_type(f, jnp.int32)
    return jnp.where(u >= 0, u, u ^ jnp.int32(0x7FFFFFFF))


def _cmp_before(ka, ia, kb, ib):
    return (ka > kb) | ((ka == kb) & (ia < ib))


def _sort_body(k, skeys_ref, sidx_ref, ei_ref, cutk_ref, cuti_ref):
    keys = skeys_ref[:, :_NSURV]
    idxs = sidx_ref[:, :_NSURV]
    E = keys.shape[0]
    pos = jax.lax.broadcasted_iota(jnp.int32, (E, _NSURV), 1)
    size = 2
    while size <= _NSURV:
        want_desc = (pos & size) == 0 if size < _NSURV else (pos >= 0)
        j = size // 2
        while j >= 1:
            self_first = (pos & j) == 0
            pk = jnp.where(self_first, jnp.roll(keys, -j, axis=1),
                           jnp.roll(keys, j, axis=1))
            pi = jnp.where(self_first, jnp.roll(idxs, -j, axis=1),
                           jnp.roll(idxs, j, axis=1))
            winner_is_self = _cmp_before(keys, idxs, pk, pi)
            take_self = (want_desc == self_first) == winner_is_self
            keys = jnp.where(take_self, keys, pk)
            idxs = jnp.where(take_self, idxs, pi)
            j //= 2
        size *= 2
    ei_ref[...] = idxs[:, :k]
    cutk_ref[...] = jnp.broadcast_to(keys[:, k - 1:k], (E, 128))
    cuti_ref[...] = jnp.broadcast_to(idxs[:, k - 1:k], (E, 128))


def _sort_survivors(skeys, sidx, E, k):
    return pl.pallas_call(
        functools.partial(_sort_body, k),
        in_specs=[
            pl.BlockSpec((E, _SPAD), lambda: (0, 0)),
            pl.BlockSpec((E, _SPAD), lambda: (0, 0)),
        ],
        out_specs=[
            pl.BlockSpec((E, k), lambda: (0, 0)),
            pl.BlockSpec((E, 128), lambda: (0, 0)),
            pl.BlockSpec((E, 128), lambda: (0, 0)),
        ],
        out_shape=[
            jax.ShapeDtypeStruct((E, k), jnp.int32),
            jax.ShapeDtypeStruct((E, 128), jnp.int32),
            jax.ShapeDtypeStruct((E, 128), jnp.int32),
        ],
    )(skeys, sidx)


def _mask_body(k, logits_ref, cutk_ref, cuti_ref, mask_ref, loss_ref,
               acc_ref):
    i = pl.program_id(0)
    E, BT = logits_ref.shape
    key = _mono_key_tc(logits_ref[...])
    tok = jax.lax.broadcasted_iota(jnp.int32, (E, BT), 1) + i * BT
    cutk = cutk_ref[:, :1]
    cuti = cuti_ref[:, :1]
    sel = (key > cutk) | ((key == cutk) & (tok <= cuti))
    m = jnp.where(sel, jnp.float32(1.0), jnp.float32(0.0))
    mask_ref[...] = m.T

    @pl.when(i == 0)
    def _():
        acc_ref[...] = jnp.zeros_like(acc_ref)

    acc_ref[0:1, :E] += jnp.sum(m, axis=1)[None, :]

    @pl.when(i == pl.num_programs(0) - 1)
    def _():
        load = acc_ref[0:1, :E]
        loss = jnp.mean(load * jnp.log(load / jnp.mean(load)))
        loss_ref[...] = jnp.full_like(loss_ref, loss)


def _dispatch_mask(logitsT, cutk, cuti, T, E, k):
    mask, loss = pl.pallas_call(
        functools.partial(_mask_body, k),
        grid=(T // _BD,),
        in_specs=[
            pl.BlockSpec((E, _BD), lambda i: (0, i)),
            pl.BlockSpec((E, 128), lambda i: (0, 0)),
            pl.BlockSpec((E, 128), lambda i: (0, 0)),
        ],
        out_specs=[
            pl.BlockSpec((_BD, E), lambda i: (i, 0)),
            pl.BlockSpec((8, 128), lambda i: (0, 0)),
        ],
        out_shape=[
            jax.ShapeDtypeStruct((T, E), jnp.float32),
            jax.ShapeDtypeStruct((8, 128), jnp.float32),
        ],
        scratch_shapes=[pltpu.VMEM((8, 128), jnp.float32)],
    )(logitsT, cutk, cuti)
    return mask, loss[0, 0]


def _mono_key_sc(f):
    u = jax.lax.bitcast_convert_type(f, jnp.int32)
    return jnp.where(u >= 0, u, u ^ jnp.int32(0x7FFFFFFF))


def _select_body(T, E, k, logits_hbm, skeys_hbm, sidx_hbm,
                 row_v, keys_v, hist_v, skeys_v, sidx_v):
    nvec = T // 16
    nhist = 2048
    lane = jax.lax.iota(jnp.int32, 16)
    ones = jnp.ones((16,), jnp.int32)
    zeros = jnp.zeros((16,), jnp.int32)
    wid = jax.lax.axis_index("s") * 2 + jax.lax.axis_index("c")

    def hist_pass(bin_mask_fn, build_keys=False):
        def z(i, c):
            for u in range(4):
                hist_v[pl.ds((i * 4 + u) * 16, 16)] = zeros
            return c
        jax.lax.fori_loop(0, nhist // 4, z, 0)

        def h(i, c):
            for u in range(4):
                base = (i * 4 + u) * 16
                if build_keys:
                    key = _mono_key_sc(row_v[pl.ds(base, 16)])
                    keys_v[pl.ds(base, 16)] = key
                else:
                    key = keys_v[pl.ds(base, 16)]
                b, m = bin_mask_fn(key)
                plsc.addupdate_scatter(hist_v, [lane * nhist + b], ones,
                                       mask=m)
            return c
        jax.lax.fori_loop(0, nvec // 4, h, 0)

    def find_bucket(rem):
        def body(jj, carry):
            above, bstar, cnt_at = carry
            j = 127 - jj
            tot = zeros
            for l in range(16):
                tot = tot + hist_v[pl.ds(l * nhist + j * 16, 16)]
            suf = jax.lax.rev(plsc.cumsum(jax.lax.rev(tot, (0,))), (0,))
            cum_gt = above + suf - tot
            hit = (cum_gt < rem) & (cum_gt + tot >= rem)
            hitn = jax.lax.reduce_max(jnp.where(hit, ones, zeros), axes=(0,))
            bsel = jax.lax.reduce_max(
                jnp.where(hit, j * 16 + lane, jnp.full((16,), -1, jnp.int32)),
                axes=(0,))
            csel = jax.lax.reduce_max(
                jnp.where(hit, cum_gt, jnp.full((16,), -1, jnp.int32)),
                axes=(0,))
            bstar = jnp.where(hitn > 0, bsel, bstar)
            cnt_at = jnp.where(hitn > 0, csel, cnt_at)
            above = above + jax.lax.reduce_sum(tot, axes=(0,))
            return above, bstar, cnt_at
        _, bstar, cnt_at = jax.lax.fori_loop(
            0, 128, body, (jnp.int32(0), jnp.int32(-1), jnp.int32(0)))
        return bstar, cnt_at

    for e in range(2):
        r = wid * 2 + e
        pltpu.sync_copy(logits_hbm.at[r], row_v)

        hist_pass(lambda key: ((key >> 21) + 1024, None), build_keys=True)
        b1, c1 = find_bucket(jnp.int32(k))
        hist_pass(lambda key: ((key >> 10) & jnp.int32(0x7FF),
                               ((key >> 21) + 1024) == b1))
        b2, c2 = find_bucket(jnp.int32(k) - c1)
        pref = ((b1 - 1024) << 11) | b2
        hist_pass(lambda key: (key & jnp.int32(0x3FF), (key >> 10) == pref))
        b3, _ = find_bucket(jnp.int32(k) - c1 - c2)
        t = ((b1 - 1024) << 21) | (b2 << 10) | b3

        def zpad(i, c):
            skeys_v[pl.ds(i * 16, 16)] = jnp.full((16,), _PAD_KEY, jnp.int32)
            sidx_v[pl.ds(i * 16, 16)] = jnp.full((16,), _PAD_IDX, jnp.int32)
            return c
        jax.lax.fori_loop(0, _SPAD // 16, zpad, 0)

        def comp(i, off):
            for u in range(4):
                base = (i * 4 + u) * 16
                key = keys_v[pl.ds(base, 16)]
                m = key >= t
                cnt = jax.lax.reduce_sum(jnp.where(m, ones, zeros), axes=(0,))
                off_c = off

                @pl.when(off_c < _NSURV)
                def _():
                    plsc.store_compressed(skeys_v.at[pl.ds(off_c, 16)], key,
                                          mask=m)
                    plsc.store_compressed(sidx_v.at[pl.ds(off_c, 16)],
                                          base + lane, mask=m)
                off = off + cnt
            return off
        jax.lax.fori_loop(0, nvec // 4, comp, jnp.int32(0))

        pltpu.sync_copy(skeys_v, skeys_hbm.at[r])
        pltpu.sync_copy(sidx_v, sidx_hbm.at[r])


def _select_survivors_sc(logitsT, E, T, k):
    mesh = plsc.VectorSubcoreMesh(core_axis_name="c", subcore_axis_name="s")
    f = pl.kernel(
        functools.partial(_select_body, T, E, k),
        out_type=[
            jax.ShapeDtypeStruct((E, _SPAD), jnp.int32),
            jax.ShapeDtypeStruct((E, _SPAD), jnp.int32),
        ],
        mesh=mesh,
        scratch_types=[
            pltpu.VMEM((T,), jnp.float32),
            pltpu.VMEM((T,), jnp.int32),
            pltpu.VMEM((T,), jnp.int32),
            pltpu.VMEM((_SPAD,), jnp.int32),
            pltpu.VMEM((_SPAD,), jnp.int32),
        ],
        compiler_params=pltpu.CompilerParams(needs_layout_passes=False),
    )
    return f(logitsT)


def kernel(x, W):
    T, H = x.shape
    E = W.shape[0]
    k = int(T * _CAPACITY_FACTOR / E)
    logitsT = _logits_t(x, W, T, H, E)
    skeys, sidx = _select_survivors_sc(logitsT, E, T, k)
    expert_indices, cutk, cuti = _sort_survivors(skeys, sidx, E, k)
    dispatch_mask, loss = _dispatch_mask(logitsT, cutk, cuti, T, E, k)
    return expert_indices, dispatch_mask, loss


# speedup vs baseline: 5.7990x; 1.1489x over previous
import functools

import jax
import jax.numpy as jnp
from jax.experimental import pallas as pl
from jax.experimental.pallas import tpu as pltpu
from jax.experimental.pallas import tpu_sc as plsc

_CAPACITY_FACTOR = 1.25
_BT = 2048
_BD = 4096
_NSURV = 1024
_SPAD = 1040
_PAD_KEY = -2147483648
_PAD_IDX = 0x7FFFFFF


def _matmul_body(x_ref, w_ref, out_ref):
    out_ref[...] = jax.lax.dot_general(
        w_ref[...], x_ref[...],
        dimension_numbers=(((1,), (1,)), ((), ())),
        preferred_element_type=jnp.float32,
    )


def _logits_t(x, W, T, H, E):
    return pl.pallas_call(
        _matmul_body,
        grid=(T // _BT,),
        in_specs=[
            pl.BlockSpec((_BT, H), lambda i: (i, 0)),
            pl.BlockSpec((E, H), lambda i: (0, 0)),
        ],
        out_specs=pl.BlockSpec((E, _BT), lambda i: (0, i)),
        out_shape=jax.ShapeDtypeStruct((E, T), jnp.float32),
    )(x, W)


def _mono_key_tc(f):
    u = jax.lax.bitcast_convert_type(f, jnp.int32)
    return jnp.where(u >= 0, u, u ^ jnp.int32(0x7FFFFFFF))


def _cmp_before(ka, ia, kb, ib):
    return (ka > kb) | ((ka == kb) & (ia < ib))


def _sort_body(k, skeys_ref, sidx_ref, ei_ref, cutk_ref, cuti_ref):
    keys = skeys_ref[:, :_NSURV]
    idxs = sidx_ref[:, :_NSURV]
    E = keys.shape[0]
    pos = jax.lax.broadcasted_iota(jnp.int32, (E, _NSURV), 1)
    size = 2
    while size <= _NSURV:
        want_desc = (pos & size) == 0 if size < _NSURV else (pos >= 0)
        j = size // 2
        while j >= 1:
            self_first = (pos & j) == 0
            pk = jnp.where(self_first, jnp.roll(keys, -j, axis=1),
                           jnp.roll(keys, j, axis=1))
            pi = jnp.where(self_first, jnp.roll(idxs, -j, axis=1),
                           jnp.roll(idxs, j, axis=1))
            winner_is_self = _cmp_before(keys, idxs, pk, pi)
            take_self = (want_desc == self_first) == winner_is_self
            keys = jnp.where(take_self, keys, pk)
            idxs = jnp.where(take_self, idxs, pi)
            j //= 2
        size *= 2
    ei_ref[...] = idxs[:, :k]
    cutk_ref[...] = jnp.broadcast_to(keys[:, k - 1:k], (E, 128))
    cuti_ref[...] = jnp.broadcast_to(idxs[:, k - 1:k], (E, 128))


def _sort_survivors(skeys, sidx, E, k):
    return pl.pallas_call(
        functools.partial(_sort_body, k),
        in_specs=[
            pl.BlockSpec((E, _SPAD), lambda: (0, 0)),
            pl.BlockSpec((E, _SPAD), lambda: (0, 0)),
        ],
        out_specs=[
            pl.BlockSpec((E, k), lambda: (0, 0)),
            pl.BlockSpec((E, 128), lambda: (0, 0)),
            pl.BlockSpec((E, 128), lambda: (0, 0)),
        ],
        out_shape=[
            jax.ShapeDtypeStruct((E, k), jnp.int32),
            jax.ShapeDtypeStruct((E, 128), jnp.int32),
            jax.ShapeDtypeStruct((E, 128), jnp.int32),
        ],
    )(skeys, sidx)


def _mask_body(k, logits_ref, cutk_ref, cuti_ref, mask_ref, loss_ref,
               acc_ref):
    i = pl.program_id(0)
    E, BT = logits_ref.shape
    key = _mono_key_tc(logits_ref[...])
    tok = jax.lax.broadcasted_iota(jnp.int32, (E, BT), 1) + i * BT
    cutk = cutk_ref[:, :1]
    cuti = cuti_ref[:, :1]
    sel = (key > cutk) | ((key == cutk) & (tok <= cuti))
    m = jnp.where(sel, jnp.float32(1.0), jnp.float32(0.0))
    mask_ref[...] = m.T

    @pl.when(i == 0)
    def _():
        acc_ref[...] = jnp.zeros_like(acc_ref)

    acc_ref[0:1, :E] += jnp.sum(m, axis=1)[None, :]

    @pl.when(i == pl.num_programs(0) - 1)
    def _():
        load = acc_ref[0:1, :E]
        loss = jnp.mean(load * jnp.log(load / jnp.mean(load)))
        loss_ref[...] = jnp.full_like(loss_ref, loss)


def _dispatch_mask(logitsT, cutk, cuti, T, E, k):
    mask, loss = pl.pallas_call(
        functools.partial(_mask_body, k),
        grid=(T // _BD,),
        in_specs=[
            pl.BlockSpec((E, _BD), lambda i: (0, i)),
            pl.BlockSpec((E, 128), lambda i: (0, 0)),
            pl.BlockSpec((E, 128), lambda i: (0, 0)),
        ],
        out_specs=[
            pl.BlockSpec((_BD, E), lambda i: (i, 0)),
            pl.BlockSpec((8, 128), lambda i: (0, 0)),
        ],
        out_shape=[
            jax.ShapeDtypeStruct((T, E), jnp.float32),
            jax.ShapeDtypeStruct((8, 128), jnp.float32),
        ],
        scratch_shapes=[pltpu.VMEM((8, 128), jnp.float32)],
    )(logitsT, cutk, cuti)
    return mask, loss[0, 0]


def _mono_key_sc(f):
    u = jax.lax.bitcast_convert_type(f, jnp.int32)
    return jnp.where(u >= 0, u, u ^ jnp.int32(0x7FFFFFFF))


def _select_body(T, E, k, logits_hbm, skeys_hbm, sidx_hbm,
                 row_v, keys_v, hist_v, skeys_v, sidx_v):
    nvec = T // 16
    nhist = 2048
    lane = jax.lax.iota(jnp.int32, 16)
    ones = jnp.ones((16,), jnp.int32)
    zeros = jnp.zeros((16,), jnp.int32)
    wid = jax.lax.axis_index("s") * 2 + jax.lax.axis_index("c")

    def hist_pass(bin_mask_fn, build_keys=False):
        def z(i, c):
            for u in range(4):
                hist_v[pl.ds((i * 4 + u) * 16, 16)] = zeros
            return c
        jax.lax.fori_loop(0, nhist // 4, z, 0)

        def h(i, c):
            for u in range(4):
                base = (i * 4 + u) * 16
                if build_keys:
                    key = _mono_key_sc(row_v[pl.ds(base, 16)])
                    keys_v[pl.ds(base, 16)] = key
                else:
                    key = keys_v[pl.ds(base, 16)]
                b, m = bin_mask_fn(key)
                plsc.addupdate_scatter(hist_v, [lane * nhist + b], ones,
                                       mask=m)
            return c
        jax.lax.fori_loop(0, nvec // 4, h, 0)

    def find_bucket(rem):
        def body(jj, carry):
            above, bstar, cnt_at = carry
            j = 127 - jj
            tot = zeros
            for l in range(16):
                tot = tot + hist_v[pl.ds(l * nhist + j * 16, 16)]
            suf = jax.lax.rev(plsc.cumsum(jax.lax.rev(tot, (0,))), (0,))
            cum_gt = above + suf - tot
            hit = (cum_gt < rem) & (cum_gt + tot >= rem)
            hitn = jax.lax.reduce_max(jnp.where(hit, ones, zeros), axes=(0,))
            bsel = jax.lax.reduce_max(
                jnp.where(hit, j * 16 + lane, jnp.full((16,), -1, jnp.int32)),
                axes=(0,))
            csel = jax.lax.reduce_max(
                jnp.where(hit, cum_gt, jnp.full((16,), -1, jnp.int32)),
                axes=(0,))
            bstar = jnp.where(hitn > 0, bsel, bstar)
            cnt_at = jnp.where(hitn > 0, csel, cnt_at)
            above = above + jax.lax.reduce_sum(tot, axes=(0,))
            return above, bstar, cnt_at
        _, bstar, cnt_at = jax.lax.fori_loop(
            0, 128, body, (jnp.int32(0), jnp.int32(-1), jnp.int32(0)))
        return bstar, cnt_at

    for e in range(2):
        r = wid * 2 + e
        pltpu.sync_copy(logits_hbm.at[r], row_v)

        hist_pass(lambda key: ((key >> 21) + 1024, None), build_keys=True)
        b1, c1 = find_bucket(jnp.int32(k))
        hist_pass(lambda key: ((key >> 10) & jnp.int32(0x7FF),
                               ((key >> 21) + 1024) == b1))
        b2, c2 = find_bucket(jnp.int32(k) - c1)
        t22 = ((b1 - 1024) << 11) | b2

        def zpad(i, c):
            skeys_v[pl.ds(i * 16, 16)] = jnp.full((16,), _PAD_KEY, jnp.int32)
            sidx_v[pl.ds(i * 16, 16)] = jnp.full((16,), _PAD_IDX, jnp.int32)
            return c
        jax.lax.fori_loop(0, _SPAD // 16, zpad, 0)

        def comp(i, off):
            for u in range(4):
                base = (i * 4 + u) * 16
                key = keys_v[pl.ds(base, 16)]
                m = (key >> 10) >= t22
                cnt = jax.lax.reduce_sum(jnp.where(m, ones, zeros), axes=(0,))
                off_c = off

                @pl.when(off_c < _NSURV)
                def _():
                    plsc.store_compressed(skeys_v.at[pl.ds(off_c, 16)], key,
                                          mask=m)
                    plsc.store_compressed(sidx_v.at[pl.ds(off_c, 16)],
                                          base + lane, mask=m)
                off = off + cnt
            return off
        jax.lax.fori_loop(0, nvec // 4, comp, jnp.int32(0))

        pltpu.sync_copy(skeys_v, skeys_hbm.at[r])
        pltpu.sync_copy(sidx_v, sidx_hbm.at[r])


def _select_survivors_sc(logitsT, E, T, k):
    mesh = plsc.VectorSubcoreMesh(core_axis_name="c", subcore_axis_name="s")
    f = pl.kernel(
        functools.partial(_select_body, T, E, k),
        out_type=[
            jax.ShapeDtypeStruct((E, _SPAD), jnp.int32),
            jax.ShapeDtypeStruct((E, _SPAD), jnp.int32),
        ],
        mesh=mesh,
        scratch_types=[
            pltpu.VMEM((T,), jnp.float32),
            pltpu.VMEM((T,), jnp.int32),
            pltpu.VMEM((T,), jnp.int32),
            pltpu.VMEM((_SPAD,), jnp.int32),
            pltpu.VMEM((_SPAD,), jnp.int32),
        ],
        compiler_params=pltpu.CompilerParams(needs_layout_passes=False),
    )
    return f(logitsT)


def kernel(x, W):
    T, H = x.shape
    E = W.shape[0]
    k = int(T * _CAPACITY_FACTOR / E)
    logitsT = _logits_t(x, W, T, H, E)
    skeys, sidx = _select_survivors_sc(logitsT, E, T, k)
    expert_indices, cutk, cuti = _sort_survivors(skeys, sidx, E, k)
    dispatch_mask, loss = _dispatch_mask(logitsT, cutk, cuti, T, E, k)
    return expert_indices, dispatch_mask, loss
